# Initial kernel scaffold; baseline (speedup 1.0000x reference)
import jax
import jax.numpy as jnp
from jax.experimental import pallas as pl


def kernel(x, attn):
    raise NotImplementedError("write your pallas kernel here")


# speedup vs baseline: 4.1418x; 4.1418x over previous
import functools

import jax
import jax.numpy as jnp
import numpy as np
from jax.experimental import pallas as pl
from jax.experimental.pallas import tpu as pltpu

N = 65536
D = 512
K = 16384
BN = 2048
LANES = 128
SUBL = N // LANES

_MININT = np.int32(-(2 ** 31))


def _sortable_key(f32val):
    b = jax.lax.bitcast_convert_type(f32val, jnp.int32)
    return jnp.where(b < 0, jnp.bitwise_xor(jnp.bitwise_not(b), _MININT), b)


def _mask_body(attn2d_ref, attn_col_ref, x_ref, o_ref, sel_ref):
    step = pl.program_id(0)

    @pl.when(step == 0)
    def _select():
        key = _sortable_key(attn2d_ref[...])

        prefix = _MININT
        cnt = jnp.sum((key >= np.int32(0)).astype(jnp.int32))
        prefix = jnp.where(cnt >= K, np.int32(0), prefix)
        for b in range(30, -1, -1):
            cand = prefix + np.int32(1 << b)
            cnt = jnp.sum((key >= cand).astype(jnp.int32))
            prefix = jnp.where(cnt >= K, cand, prefix)
        t_key = prefix

        c_gt = jnp.sum((key > t_key).astype(jnp.int32))
        budget = np.int32(K) - c_gt
        tied = key == t_key
        row = jax.lax.broadcasted_iota(jnp.int32, (SUBL, LANES), 0)
        col = jax.lax.broadcasted_iota(jnp.int32, (SUBL, LANES), 1)
        idx = row * LANES + col
        ipfx = np.int32(0)
        for b in range(15, -1, -1):
            test = ipfx + np.int32((1 << b) - 1)
            c = jnp.sum((tied & (idx <= test)).astype(jnp.int32))
            ipfx = jnp.where(c >= budget, ipfx, ipfx + np.int32(1 << b))
        i_star = ipfx

        sel_ref[0] = t_key
        sel_ref[1] = i_star

    t_key = sel_ref[0]
    i_star = sel_ref[1]
    keyc = _sortable_key(attn_col_ref[...])
    idxc = jax.lax.broadcasted_iota(jnp.int32, (BN, 1), 0) + step * BN
    masked = (keyc > t_key) | ((keyc == t_key) & (idxc <= i_star))
    keep = jnp.where(masked, np.float32(0.0), np.float32(1.0))
    o_ref[...] = x_ref[...] * keep


@jax.jit
def kernel(x, attn):
    x2 = x.reshape(N, D)
    attn2d = attn.reshape(SUBL, LANES)
    attn_col = attn.reshape(N, 1)

    out = pl.pallas_call(
        _mask_body,
        grid=(N // BN,),
        in_specs=[
            pl.BlockSpec((SUBL, LANES), lambda i: (0, 0)),
            pl.BlockSpec((BN, 1), lambda i: (i, 0)),
            pl.BlockSpec((BN, D), lambda i: (i, 0)),
        ],
        out_specs=pl.BlockSpec((BN, D), lambda i: (i, 0)),
        out_shape=jax.ShapeDtypeStruct((N, D), jnp.float32),
        scratch_shapes=[pltpu.SMEM((2,), jnp.int32)],
        compiler_params=pltpu.CompilerParams(
            dimension_semantics=("arbitrary",),
        ),
    )(attn2d, attn_col, x2)
    return out.reshape(1, N, D)


# speedup vs baseline: 5.4972x; 1.3272x over previous
import jax
import jax.numpy as jnp
import numpy as np
from jax.experimental import pallas as pl
from jax.experimental.pallas import tpu as pltpu

N = 65536
D = 512
K = 16384
LANES = 128
SUBL = N // LANES
R = 32
BN = R * LANES

_MININT = np.int32(-(2 ** 31))


def _sortable_key(f32val):
    b = jax.lax.bitcast_convert_type(f32val, jnp.int32)
    return jnp.where(b < 0, jnp.bitwise_xor(jnp.bitwise_not(b), _MININT), b)


def _mask_body(attn2d_ref, x_ref, o_ref, keep_ref):
    step = pl.program_id(0)

    @pl.when(step == 0)
    def _select():
        key = _sortable_key(attn2d_ref[...])

        prefix = _MININT
        cnt = jnp.sum((key >= np.int32(0)).astype(jnp.int32))
        prefix = jnp.where(cnt >= K, np.int32(0), prefix)
        for b in range(30, -1, -1):
            cand = prefix + np.int32(1 << b)
            cnt = jnp.sum((key >= cand).astype(jnp.int32))
            prefix = jnp.where(cnt >= K, cand, prefix)
        t_key = prefix

        c_gt = jnp.sum((key > t_key).astype(jnp.int32))
        budget = np.int32(K) - c_gt
        tied = key == t_key
        row = jax.lax.broadcasted_iota(jnp.int32, (SUBL, LANES), 0)
        col = jax.lax.broadcasted_iota(jnp.int32, (SUBL, LANES), 1)
        idx = row * LANES + col
        ipfx = np.int32(0)
        for b in range(15, -1, -1):
            test = ipfx + np.int32((1 << b) - 1)
            c = jnp.sum((tied & (idx <= test)).astype(jnp.int32))
            ipfx = jnp.where(c >= budget, ipfx, ipfx + np.int32(1 << b))
        i_star = ipfx

        masked = (key > t_key) | (tied & (idx <= i_star))
        keep_ref[...] = jnp.where(masked, np.float32(0.0), np.float32(1.0))

    keep = keep_ref[pl.ds(step * R, R), :]
    o_ref[...] = x_ref[...] * keep[:, :, None]


@jax.jit
def kernel(x, attn):
    x3 = x.reshape(SUBL, LANES, D)
    attn2d = attn.reshape(SUBL, LANES)

    out = pl.pallas_call(
        _mask_body,
        grid=(SUBL // R,),
        in_specs=[
            pl.BlockSpec((SUBL, LANES), lambda i: (0, 0)),
            pl.BlockSpec((R, LANES, D), lambda i: (i, 0, 0)),
        ],
        out_specs=pl.BlockSpec((R, LANES, D), lambda i: (i, 0, 0)),
        out_shape=jax.ShapeDtypeStruct((SUBL, LANES, D), jnp.float32),
        scratch_shapes=[pltpu.VMEM((SUBL, LANES), jnp.float32)],
        compiler_params=pltpu.CompilerParams(
            dimension_semantics=("arbitrary",),
        ),
    )(attn2d, x3)
    return out.reshape(1, N, D)
